# baseline (speedup 1.0000x reference)
import functools

import jax
import jax.numpy as jnp
from jax import lax
from jax.experimental import pallas as pl
from jax.experimental.pallas import tpu as pltpu
from jax.experimental.pallas import tpu_sc as plsc

import numpy as np

B, T, N, D, H, MN = 32, 20, 32, 128, 64, 16
TK, SK = 8, 8
J = 1 + MN
SUB = TK * (1 + SK)
ROWS = B * N * SUB
TBL = B * T * N
SIGMA_MIN, SIGMA_MAX = -10.0, 2.0

_EPS_T = np.asarray(
    jax.random.normal(jax.random.key(42), (B, N, T, J), dtype=jnp.float32)
).transpose(0, 1, 3, 2).copy()


def _feat_body(s_ref, wf_ref, bf_ref, hraw_ref, stats_ref):
    b = pl.program_id(0)
    x = s_ref[0].reshape(T * N, D)
    hr = jnp.dot(x.astype(jnp.bfloat16), wf_ref[...].astype(jnp.bfloat16),
                 preferred_element_type=jnp.float32) + bf_ref[...]
    hraw_ref[0] = hr.reshape(T, N, H)

    @pl.when(b == 0)
    def _():
        stats_ref[...] = jnp.zeros_like(stats_ref)

    stats_ref[0:1] += jnp.sum(hr, axis=0, keepdims=True)
    stats_ref[1:2] += jnp.sum(hr * hr, axis=0, keepdims=True)


def _stage_a(s, W_feat, bf):
    return pl.pallas_call(
        _feat_body,
        grid=(B,),
        in_specs=[
            pl.BlockSpec((1, T, N, D), lambda b: (b, 0, 0, 0)),
            pl.BlockSpec((D, H), lambda b: (0, 0)),
            pl.BlockSpec((1, H), lambda b: (0, 0)),
        ],
        out_specs=[
            pl.BlockSpec((1, T, N, H), lambda b: (b, 0, 0, 0)),
            pl.BlockSpec((8, H), lambda b: (0, 0)),
        ],
        out_shape=[
            jax.ShapeDtypeStruct((B, T, N, H), jnp.float32),
            jax.ShapeDtypeStruct((8, H), jnp.float32),
        ],
    )(s, W_feat, bf)


def _score_body(hraw_ref, kflat_ref, stats_ref, gin_ref, bin_ref, wml_ref,
                bml_ref, mu_ref, lv_ref, st2_ref):
    b = pl.program_id(0)
    cnt = float(B * T * N)
    mean = stats_ref[0:1] / cnt
    var = stats_ref[1:2] / cnt - mean * mean
    std = jnp.sqrt(var + 1e-5)

    hr = hraw_ref[0].reshape(T * N, H)
    hbn = (hr - mean) / std * gin_ref[...] + bin_ref[...]
    hml = jnp.dot(hbn.astype(jnp.bfloat16), wml_ref[...].astype(jnp.bfloat16),
                  preferred_element_type=jnp.float32) + bml_ref[...]
    hml3 = hml.reshape(T, N, 2)
    hmuT = hml3[:, :, 0].T
    hlvT = hml3[:, :, 1].T

    kflat = kflat_ref[0]
    onehot = (kflat == lax.broadcasted_iota(jnp.int32, (N * MN, N), 1)
              ).astype(jnp.float32)
    mu_nei = jnp.dot(onehot, hmuT, preferred_element_type=jnp.float32).reshape(N, MN, T)
    lv_nei = jnp.dot(onehot, hlvT, preferred_element_type=jnp.float32).reshape(N, MN, T)
    mu_raw = jnp.concatenate([hmuT.reshape(N, 1, T), mu_nei], axis=1)
    lv_raw = jnp.concatenate([hlvT.reshape(N, 1, T), lv_nei], axis=1)
    mu_ref[0] = mu_raw
    lv_ref[0] = lv_raw

    @pl.when(b == 0)
    def _():
        st2_ref[...] = jnp.zeros_like(st2_ref)

    st2_ref[:, 0:1] += jnp.sum(jnp.sum(mu_raw, axis=2), axis=1, keepdims=True)
    st2_ref[:, 1:2] += jnp.sum(jnp.sum(mu_raw * mu_raw, axis=2), axis=1, keepdims=True)
    st2_ref[:, 2:3] += jnp.sum(jnp.sum(lv_raw, axis=2), axis=1, keepdims=True)
    st2_ref[:, 3:4] += jnp.sum(jnp.sum(lv_raw * lv_raw, axis=2), axis=1, keepdims=True)


def _stage_b(hraw, kflat, stats, gin, bin_, wml, bml):
    return pl.pallas_call(
        _score_body,
        grid=(B,),
        in_specs=[
            pl.BlockSpec((1, T, N, H), lambda b: (b, 0, 0, 0)),
            pl.BlockSpec((1, N * MN, 1), lambda b: (b, 0, 0)),
            pl.BlockSpec((8, H), lambda b: (0, 0)),
            pl.BlockSpec((1, H), lambda b: (0, 0)),
            pl.BlockSpec((1, H), lambda b: (0, 0)),
            pl.BlockSpec((H, 2), lambda b: (0, 0)),
            pl.BlockSpec((1, 2), lambda b: (0, 0)),
        ],
        out_specs=[
            pl.BlockSpec((1, N, J, T), lambda b: (b, 0, 0, 0)),
            pl.BlockSpec((1, N, J, T), lambda b: (b, 0, 0, 0)),
            pl.BlockSpec((N, 8), lambda b: (0, 0)),
        ],
        out_shape=[
            jax.ShapeDtypeStruct((B, N, J, T), jnp.float32),
            jax.ShapeDtypeStruct((B, N, J, T), jnp.float32),
            jax.ShapeDtypeStruct((N, 8), jnp.float32),
        ],
    )(hraw, kflat, stats, gin, bin_, wml, bml)


def _select_body(mu_ref, lv_ref, st2_ref, gmu_ref, bmu_ref, glv_ref, blv_ref,
                 eps_ref, nnei_ref, knei_ref, gidx_ref):
    b = pl.program_id(0)
    cntn = float(B * T * J)
    msum = st2_ref[:, 0:1]; msq = st2_ref[:, 1:2]
    lsum = st2_ref[:, 2:3]; lsq = st2_ref[:, 3:4]
    mmean = msum / cntn
    mstd = jnp.sqrt(msq / cntn - mmean * mmean + 1e-5)
    lmean = lsum / cntn
    lstd = jnp.sqrt(lsq / cntn - lmean * lmean + 1e-5)

    mu = jnp.tanh((mu_ref[0] - mmean.reshape(N, 1, 1)) / mstd.reshape(N, 1, 1)
                  * gmu_ref[...].reshape(N, 1, 1) + bmu_ref[...].reshape(N, 1, 1))
    logvar = jnp.clip((lv_ref[0] - lmean.reshape(N, 1, 1)) / lstd.reshape(N, 1, 1)
                      * glv_ref[...].reshape(N, 1, 1) + blv_ref[...].reshape(N, 1, 1),
                      SIGMA_MIN, SIGMA_MAX)
    dis = eps_ref[0] * jnp.exp(logvar) + mu

    t_sc = dis[:, 0, :]
    s_sc = dis[:, 1:, :]
    mrange = lax.broadcasted_iota(jnp.int32, (N, MN, T), 1)
    nn = nnei_ref[0].reshape(N, 1, 1)
    s_sc = jnp.where(mrange < nn, s_sc, -1e9)

    iota_t = lax.broadcasted_iota(jnp.int32, (N, T), 1)
    iota_t3 = lax.broadcasted_iota(jnp.int32, (N, MN, T), 2)
    iota_m = lax.broadcasted_iota(jnp.int32, (N, MN), 1)
    n_ids = lax.broadcasted_iota(jnp.int32, (N, 1), 0)
    kneif = knei_ref[0].astype(jnp.float32)
    neg_inf = jnp.float32(-jnp.inf)

    ts = t_sc
    tsels = []
    for _ in range(TK):
        mx = jnp.max(ts, axis=1, keepdims=True)
        idx = jnp.min(jnp.where(ts >= mx, iota_t, T), axis=1, keepdims=True)
        tsels.append(idx)
        ts = jnp.where(iota_t == idx, neg_inf, ts)

    cols = []
    for ti in range(TK):
        tsel = tsels[ti]
        rowbase = (b * T + tsel) * N
        cols.append(rowbase + n_ids)
        sel_t = (iota_t3 == tsel.reshape(N, 1, 1)).astype(jnp.float32)
        ss = jnp.sum(s_sc * sel_t, axis=2)
        for _ in range(SK):
            mx = jnp.max(ss, axis=1, keepdims=True)
            midx = jnp.min(jnp.where(ss >= mx, iota_m, MN), axis=1, keepdims=True)
            hit = iota_m == midx
            ss = jnp.where(hit, neg_inf, ss)
            ksel = jnp.sum(jnp.where(hit, kneif, 0.0), axis=1, keepdims=True)
            cols.append(rowbase + ksel.astype(jnp.int32))
    gidx_ref[0] = jnp.concatenate(cols, axis=1)


def _stage_c(mu_raw, lv_raw, st2, gmu, bmu, glv, blv, eps_t, nnei3, k_nei):
    return pl.pallas_call(
        _select_body,
        grid=(B,),
        in_specs=[
            pl.BlockSpec((1, N, J, T), lambda b: (b, 0, 0, 0)),
            pl.BlockSpec((1, N, J, T), lambda b: (b, 0, 0, 0)),
            pl.BlockSpec((N, 8), lambda b: (0, 0)),
            pl.BlockSpec((N, 1), lambda b: (0, 0)),
            pl.BlockSpec((N, 1), lambda b: (0, 0)),
            pl.BlockSpec((N, 1), lambda b: (0, 0)),
            pl.BlockSpec((N, 1), lambda b: (0, 0)),
            pl.BlockSpec((1, N, J, T), lambda b: (b, 0, 0, 0)),
            pl.BlockSpec((1, N, 1), lambda b: (b, 0, 0)),
            pl.BlockSpec((1, N, MN), lambda b: (b, 0, 0)),
        ],
        out_specs=pl.BlockSpec((1, N, SUB), lambda b: (b, 0, 0)),
        out_shape=jax.ShapeDtypeStruct((B, N, SUB), jnp.int32),
    )(mu_raw, lv_raw, st2, gmu, bmu, glv, blv, eps_t, nnei3, k_nei)


_GW = 128


def _sc_gather(table, idx):
    idx2 = idx.reshape(1, ROWS)
    mesh = plsc.VectorSubcoreMesh(core_axis_name="core", subcore_axis_name="subcore")

    @functools.partial(
        pl.kernel,
        out_type=jax.ShapeDtypeStruct((ROWS, H), jnp.float32),
        mesh=mesh,
        compiler_params=pltpu.CompilerParams(use_tc_tiling_on_sc=False),
    )
    def k(x_hbm, i_hbm, o_hbm):
        def body(i_vmem, o_vmem):
            pltpu.sync_copy(x_hbm.at[i_vmem.at[0]], o_vmem)

        pltpu.emit_pipeline(
            body,
            grid=(ROWS // _GW,),
            in_specs=[pl.BlockSpec((1, _GW), index_map=lambda i: (0, i))],
            out_specs=[pl.BlockSpec((_GW, H), index_map=lambda i: (i, 0))],
            core_axis_name=("core", "subcore"),
            dimension_semantics=(pltpu.PARALLEL,),
        )(i_hbm, o_hbm)

    return k(table, idx2)


def _gnn_body(sub_ref, stats_ref, gin_ref, bin_ref, w1_ref, b1_ref, w2_ref,
              b2_ref, w3_ref, b3_ref, out_ref):
    cnt = float(B * T * N)
    mean = stats_ref[0:1] / cnt
    var = stats_ref[1:2] / cnt - mean * mean
    std = jnp.sqrt(var + 1e-5)

    x = (sub_ref[0] - mean) / std * gin_ref[...] + bin_ref[...]
    pools = []
    for w_ref, b_ref in ((w1_ref, b1_ref), (w2_ref, b2_ref), (w3_ref, b3_ref)):
        x3 = x.reshape(N, SUB, H)
        agg = jnp.mean(x3, axis=1, keepdims=True)
        xa = (x3 + agg).reshape(N * SUB, H)
        x = jnp.maximum(
            jnp.dot(xa.astype(jnp.bfloat16), w_ref[...].astype(jnp.bfloat16),
                    preferred_element_type=jnp.float32) + b_ref[...],
            0.0)
        pools.append(jnp.mean(x.reshape(N, SUB, H), axis=1))
    out_ref[0] = jnp.concatenate(pools, axis=1)


def _stage_e(sub, stats, gin, bin_, Wg1, bg1, Wg2, bg2, Wg3, bg3):
    wspec = pl.BlockSpec((H, H), lambda b: (0, 0))
    bspec = pl.BlockSpec((1, H), lambda b: (0, 0))
    return pl.pallas_call(
        _gnn_body,
        grid=(B,),
        in_specs=[
            pl.BlockSpec((1, N * SUB, H), lambda b: (b, 0, 0)),
            pl.BlockSpec((8, H), lambda b: (0, 0)),
            bspec, bspec, wspec, bspec, wspec, bspec, wspec, bspec,
        ],
        out_specs=pl.BlockSpec((1, N, 3 * H), lambda b: (b, 0, 0)),
        out_shape=jax.ShapeDtypeStruct((B, N, 3 * H), jnp.float32),
    )(sub, stats, gin, bin_, Wg1, bg1, Wg2, bg2, Wg3, bg3)


def kernel(s, k_nei, n_nei, W_feat, b_feat, g_in, beta_in, W_mu, b_mu, g_mu,
           beta_mu, W_lv, b_lv, g_lv, beta_lv, Wg1, bg1, Wg2, bg2, Wg3, bg3):
    bf = b_feat.reshape(1, H)
    gin = g_in.reshape(1, H)
    bin_ = beta_in.reshape(1, H)
    wml = jnp.concatenate([W_mu, W_lv], axis=1)
    bml = jnp.concatenate([b_mu, b_lv]).reshape(1, 2)
    gmu = g_mu.reshape(N, 1)
    bmu = beta_mu.reshape(N, 1)
    glv = g_lv.reshape(N, 1)
    blv = beta_lv.reshape(N, 1)
    eps_t = jnp.asarray(_EPS_T)
    nnei3 = n_nei.reshape(B, N, 1)
    kflat = k_nei.reshape(B, N * MN, 1)

    hraw, stats = _stage_a(s, W_feat, bf)
    mu_raw, lv_raw, st2 = _stage_b(hraw, kflat, stats, gin, bin_, wml, bml)
    gidx = _stage_c(mu_raw, lv_raw, st2, gmu, bmu, glv, blv, eps_t, nnei3, k_nei)
    sub = _sc_gather(hraw.reshape(TBL, H), gidx.reshape(ROWS))
    return _stage_e(sub.reshape(B, N * SUB, H), stats, gin, bin_,
                    Wg1, bg1.reshape(1, H), Wg2, bg2.reshape(1, H),
                    Wg3, bg3.reshape(1, H))


# speedup vs baseline: 1.0661x; 1.0661x over previous
import functools

import jax
import jax.numpy as jnp
from jax import lax
from jax.experimental import pallas as pl
from jax.experimental.pallas import tpu as pltpu
from jax.experimental.pallas import tpu_sc as plsc

import numpy as np

B, T, N, D, H, MN = 32, 20, 32, 128, 64, 16
TK, SK = 8, 8
J = 1 + MN
SUB = TK * (1 + SK)
ROWS = B * N * SUB
TBL = B * T * N
SIGMA_MIN, SIGMA_MAX = -10.0, 2.0

_EPS_T = np.asarray(
    jax.random.normal(jax.random.key(42), (B, N, T, J), dtype=jnp.float32)
).transpose(0, 1, 3, 2).copy()


BB = 4
NR = BB * N


def _feat_body(s_ref, wf_ref, bf_ref, hraw_ref, stats_ref):
    b = pl.program_id(0)
    x = s_ref[...].reshape(BB * T * N, D)
    hr = jnp.dot(x.astype(jnp.bfloat16), wf_ref[...].astype(jnp.bfloat16),
                 preferred_element_type=jnp.float32) + bf_ref[...]
    hraw_ref[...] = hr.reshape(BB, T, N, H)

    @pl.when(b == 0)
    def _():
        stats_ref[...] = jnp.zeros_like(stats_ref)

    stats_ref[0:1] += jnp.sum(hr, axis=0, keepdims=True)
    stats_ref[1:2] += jnp.sum(hr * hr, axis=0, keepdims=True)


def _stage_a(s, W_feat, bf):
    return pl.pallas_call(
        _feat_body,
        grid=(B // BB,),
        in_specs=[
            pl.BlockSpec((BB, T, N, D), lambda b: (b, 0, 0, 0)),
            pl.BlockSpec((D, H), lambda b: (0, 0)),
            pl.BlockSpec((1, H), lambda b: (0, 0)),
        ],
        out_specs=[
            pl.BlockSpec((BB, T, N, H), lambda b: (b, 0, 0, 0)),
            pl.BlockSpec((8, H), lambda b: (0, 0)),
        ],
        out_shape=[
            jax.ShapeDtypeStruct((B, T, N, H), jnp.float32),
            jax.ShapeDtypeStruct((8, H), jnp.float32),
        ],
    )(s, W_feat, bf)


def _score_body(hraw_ref, kflat_ref, stats_ref, gin_ref, bin_ref, wml_ref,
                bml_ref, mu_ref, lv_ref, st2_ref):
    b = pl.program_id(0)
    cnt = float(B * T * N)
    mean = stats_ref[0:1] / cnt
    var = stats_ref[1:2] / cnt - mean * mean
    std = jnp.sqrt(var + 1e-5)

    hr = hraw_ref[...].reshape(BB * T * N, H)
    hbn = (hr - mean) / std * gin_ref[...] + bin_ref[...]
    hml = jnp.dot(hbn.astype(jnp.bfloat16), wml_ref[...].astype(jnp.bfloat16),
                  preferred_element_type=jnp.float32) + bml_ref[...]
    hml4 = hml.reshape(BB, T, N, 2)
    hmuT = jnp.transpose(hml4[:, :, :, 0], (0, 2, 1)).reshape(NR, T)
    hlvT = jnp.transpose(hml4[:, :, :, 1], (0, 2, 1)).reshape(NR, T)

    kflat = kflat_ref[...].reshape(BB * N * MN, 1)
    blocal = lax.broadcasted_iota(jnp.int32, (BB * N * MN, 1), 0) // (N * MN)
    kglob = kflat + blocal * N
    onehot = (kglob == lax.broadcasted_iota(jnp.int32, (BB * N * MN, NR), 1)
              ).astype(jnp.float32)
    mu_nei = jnp.dot(onehot, hmuT, preferred_element_type=jnp.float32).reshape(NR, MN, T)
    lv_nei = jnp.dot(onehot, hlvT, preferred_element_type=jnp.float32).reshape(NR, MN, T)
    mu_raw = jnp.concatenate([hmuT.reshape(NR, 1, T), mu_nei], axis=1)
    lv_raw = jnp.concatenate([hlvT.reshape(NR, 1, T), lv_nei], axis=1)
    mu_ref[...] = mu_raw.reshape(BB, N, J, T)
    lv_ref[...] = lv_raw.reshape(BB, N, J, T)

    @pl.when(b == 0)
    def _():
        st2_ref[...] = jnp.zeros_like(st2_ref)

    def _nodesum(x):
        s = jnp.sum(jnp.sum(x, axis=2), axis=1).reshape(BB, N)
        return jnp.sum(s.T, axis=1, keepdims=True)

    st2_ref[:, 0:1] += _nodesum(mu_raw)
    st2_ref[:, 1:2] += _nodesum(mu_raw * mu_raw)
    st2_ref[:, 2:3] += _nodesum(lv_raw)
    st2_ref[:, 3:4] += _nodesum(lv_raw * lv_raw)


def _stage_b(hraw, kflat, stats, gin, bin_, wml, bml):
    return pl.pallas_call(
        _score_body,
        grid=(B // BB,),
        in_specs=[
            pl.BlockSpec((BB, T, N, H), lambda b: (b, 0, 0, 0)),
            pl.BlockSpec((BB, N * MN, 1), lambda b: (b, 0, 0)),
            pl.BlockSpec((8, H), lambda b: (0, 0)),
            pl.BlockSpec((1, H), lambda b: (0, 0)),
            pl.BlockSpec((1, H), lambda b: (0, 0)),
            pl.BlockSpec((H, 2), lambda b: (0, 0)),
            pl.BlockSpec((1, 2), lambda b: (0, 0)),
        ],
        out_specs=[
            pl.BlockSpec((BB, N, J, T), lambda b: (b, 0, 0, 0)),
            pl.BlockSpec((BB, N, J, T), lambda b: (b, 0, 0, 0)),
            pl.BlockSpec((N, 8), lambda b: (0, 0)),
        ],
        out_shape=[
            jax.ShapeDtypeStruct((B, N, J, T), jnp.float32),
            jax.ShapeDtypeStruct((B, N, J, T), jnp.float32),
            jax.ShapeDtypeStruct((N, 8), jnp.float32),
        ],
    )(hraw, kflat, stats, gin, bin_, wml, bml)


def _select_body(mu_ref, lv_ref, st2_ref, gmu_ref, bmu_ref, glv_ref, blv_ref,
                 eps_ref, nnei_ref, knei_ref, gidx_ref):
    b = pl.program_id(0)
    cntn = float(B * T * J)
    msum = st2_ref[:, 0:1]; msq = st2_ref[:, 1:2]
    lsum = st2_ref[:, 2:3]; lsq = st2_ref[:, 3:4]
    mmean = msum / cntn
    mstd = jnp.sqrt(msq / cntn - mmean * mmean + 1e-5)
    lmean = lsum / cntn
    lstd = jnp.sqrt(lsq / cntn - lmean * lmean + 1e-5)

    mm = mmean.reshape(1, N, 1, 1); ms = mstd.reshape(1, N, 1, 1)
    lm = lmean.reshape(1, N, 1, 1); ls = lstd.reshape(1, N, 1, 1)
    gm = gmu_ref[...].reshape(1, N, 1, 1); bm = bmu_ref[...].reshape(1, N, 1, 1)
    gl = glv_ref[...].reshape(1, N, 1, 1); bl = blv_ref[...].reshape(1, N, 1, 1)
    mu = jnp.tanh((mu_ref[...] - mm) / ms * gm + bm)
    logvar = jnp.clip((lv_ref[...] - lm) / ls * gl + bl, SIGMA_MIN, SIGMA_MAX)
    dis = (eps_ref[...] * jnp.exp(logvar) + mu).reshape(NR, J, T)

    t_sc = dis[:, 0, :]
    s_sc = dis[:, 1:, :]
    mrange = lax.broadcasted_iota(jnp.int32, (NR, MN, T), 1)
    nn = nnei_ref[...].reshape(NR, 1, 1)
    s_sc = jnp.where(mrange < nn, s_sc, -1e9)

    iota_t = lax.broadcasted_iota(jnp.int32, (NR, T), 1)
    iota_t3 = lax.broadcasted_iota(jnp.int32, (NR, MN, T), 2)
    iota_m = lax.broadcasted_iota(jnp.int32, (NR, MN), 1)
    row_ids = lax.broadcasted_iota(jnp.int32, (NR, 1), 0)
    n_ids = row_ids % N
    b_ids = b * BB + row_ids // N
    kneif = knei_ref[...].reshape(NR, MN).astype(jnp.float32)
    neg_inf = jnp.float32(-jnp.inf)

    ts = t_sc
    tsels = []
    for _ in range(TK):
        mx = jnp.max(ts, axis=1, keepdims=True)
        idx = jnp.min(jnp.where(ts >= mx, iota_t, T), axis=1, keepdims=True)
        tsels.append(idx)
        ts = jnp.where(iota_t == idx, neg_inf, ts)

    cols = []
    for ti in range(TK):
        tsel = tsels[ti]
        rowbase = (b_ids * T + tsel) * N
        cols.append(rowbase + n_ids)
        sel_t = (iota_t3 == tsel.reshape(NR, 1, 1)).astype(jnp.float32)
        ss = jnp.sum(s_sc * sel_t, axis=2)
        for _ in range(SK):
            mx = jnp.max(ss, axis=1, keepdims=True)
            midx = jnp.min(jnp.where(ss >= mx, iota_m, MN), axis=1, keepdims=True)
            hit = iota_m == midx
            ss = jnp.where(hit, neg_inf, ss)
            ksel = jnp.sum(jnp.where(hit, kneif, 0.0), axis=1, keepdims=True)
            cols.append(rowbase + ksel.astype(jnp.int32))
    gidx_ref[...] = jnp.concatenate(cols, axis=1).reshape(BB, N, SUB)


def _stage_c(mu_raw, lv_raw, st2, gmu, bmu, glv, blv, eps_t, nnei3, k_nei):
    return pl.pallas_call(
        _select_body,
        grid=(B // BB,),
        in_specs=[
            pl.BlockSpec((BB, N, J, T), lambda b: (b, 0, 0, 0)),
            pl.BlockSpec((BB, N, J, T), lambda b: (b, 0, 0, 0)),
            pl.BlockSpec((N, 8), lambda b: (0, 0)),
            pl.BlockSpec((N, 1), lambda b: (0, 0)),
            pl.BlockSpec((N, 1), lambda b: (0, 0)),
            pl.BlockSpec((N, 1), lambda b: (0, 0)),
            pl.BlockSpec((N, 1), lambda b: (0, 0)),
            pl.BlockSpec((BB, N, J, T), lambda b: (b, 0, 0, 0)),
            pl.BlockSpec((BB, N, 1), lambda b: (b, 0, 0)),
            pl.BlockSpec((BB, N, MN), lambda b: (b, 0, 0)),
        ],
        out_specs=pl.BlockSpec((BB, N, SUB), lambda b: (b, 0, 0)),
        out_shape=jax.ShapeDtypeStruct((B, N, SUB), jnp.int32),
    )(mu_raw, lv_raw, st2, gmu, bmu, glv, blv, eps_t, nnei3, k_nei)


_GW = 128


def _sc_gather(table, idx):
    idx2 = idx.reshape(1, ROWS)
    mesh = plsc.VectorSubcoreMesh(core_axis_name="core", subcore_axis_name="subcore")

    @functools.partial(
        pl.kernel,
        out_type=jax.ShapeDtypeStruct((ROWS, H), jnp.float32),
        mesh=mesh,
        compiler_params=pltpu.CompilerParams(use_tc_tiling_on_sc=False),
    )
    def k(x_hbm, i_hbm, o_hbm):
        def body(i_vmem, o_vmem):
            pltpu.sync_copy(x_hbm.at[i_vmem.at[0]], o_vmem)

        pltpu.emit_pipeline(
            body,
            grid=(ROWS // _GW,),
            in_specs=[pl.BlockSpec((1, _GW), index_map=lambda i: (0, i))],
            out_specs=[pl.BlockSpec((_GW, H), index_map=lambda i: (i, 0))],
            core_axis_name=("core", "subcore"),
            dimension_semantics=(pltpu.PARALLEL,),
        )(i_hbm, o_hbm)

    return k(table, idx2)


def _gnn_body(sub_ref, stats_ref, gin_ref, bin_ref, w1_ref, b1_ref, w2_ref,
              b2_ref, w3_ref, b3_ref, out_ref):
    cnt = float(B * T * N)
    mean = stats_ref[0:1] / cnt
    var = stats_ref[1:2] / cnt - mean * mean
    std = jnp.sqrt(var + 1e-5)

    x = (sub_ref[...].reshape(BB * N * SUB, H)
         - mean) / std * gin_ref[...] + bin_ref[...]
    pools = []
    for w_ref, b_ref in ((w1_ref, b1_ref), (w2_ref, b2_ref), (w3_ref, b3_ref)):
        x3 = x.reshape(NR, SUB, H)
        agg = jnp.mean(x3, axis=1, keepdims=True)
        xa = (x3 + agg).reshape(NR * SUB, H)
        x = jnp.maximum(
            jnp.dot(xa.astype(jnp.bfloat16), w_ref[...].astype(jnp.bfloat16),
                    preferred_element_type=jnp.float32) + b_ref[...],
            0.0)
        pools.append(jnp.mean(x.reshape(NR, SUB, H), axis=1))
    out_ref[...] = jnp.concatenate(pools, axis=1).reshape(BB, N, 3 * H)


def _stage_e(sub, stats, gin, bin_, Wg1, bg1, Wg2, bg2, Wg3, bg3):
    wspec = pl.BlockSpec((H, H), lambda b: (0, 0))
    bspec = pl.BlockSpec((1, H), lambda b: (0, 0))
    return pl.pallas_call(
        _gnn_body,
        grid=(B // BB,),
        in_specs=[
            pl.BlockSpec((BB, N * SUB, H), lambda b: (b, 0, 0)),
            pl.BlockSpec((8, H), lambda b: (0, 0)),
            bspec, bspec, wspec, bspec, wspec, bspec, wspec, bspec,
        ],
        out_specs=pl.BlockSpec((BB, N, 3 * H), lambda b: (b, 0, 0)),
        out_shape=jax.ShapeDtypeStruct((B, N, 3 * H), jnp.float32),
    )(sub, stats, gin, bin_, Wg1, bg1, Wg2, bg2, Wg3, bg3)


def kernel(s, k_nei, n_nei, W_feat, b_feat, g_in, beta_in, W_mu, b_mu, g_mu,
           beta_mu, W_lv, b_lv, g_lv, beta_lv, Wg1, bg1, Wg2, bg2, Wg3, bg3):
    bf = b_feat.reshape(1, H)
    gin = g_in.reshape(1, H)
    bin_ = beta_in.reshape(1, H)
    wml = jnp.concatenate([W_mu, W_lv], axis=1)
    bml = jnp.concatenate([b_mu, b_lv]).reshape(1, 2)
    gmu = g_mu.reshape(N, 1)
    bmu = beta_mu.reshape(N, 1)
    glv = g_lv.reshape(N, 1)
    blv = beta_lv.reshape(N, 1)
    eps_t = jnp.asarray(_EPS_T)
    nnei3 = n_nei.reshape(B, N, 1)
    kflat = k_nei.reshape(B, N * MN, 1)

    hraw, stats = _stage_a(s, W_feat, bf)
    mu_raw, lv_raw, st2 = _stage_b(hraw, kflat, stats, gin, bin_, wml, bml)
    gidx = _stage_c(mu_raw, lv_raw, st2, gmu, bmu, glv, blv, eps_t, nnei3, k_nei)
    sub = _sc_gather(hraw.reshape(TBL, H), gidx.reshape(ROWS))
    return _stage_e(sub.reshape(B, N * SUB, H), stats, gin, bin_,
                    Wg1, bg1.reshape(1, H), Wg2, bg2.reshape(1, H),
                    Wg3, bg3.reshape(1, H))


# speedup vs baseline: 1.1565x; 1.0847x over previous
import functools

import jax
import jax.numpy as jnp
from jax import lax
from jax.experimental import pallas as pl
from jax.experimental.pallas import tpu as pltpu
from jax.experimental.pallas import tpu_sc as plsc

import numpy as np

B, T, N, D, H, MN = 32, 20, 32, 128, 64, 16
TK, SK = 8, 8
J = 1 + MN
SUB = TK * (1 + SK)
ROWS = B * N * SUB
TBL = B * T * N
SIGMA_MIN, SIGMA_MAX = -10.0, 2.0

_EPS_T = np.asarray(
    jax.random.normal(jax.random.key(42), (B, N, T, J), dtype=jnp.float32)
).transpose(0, 1, 3, 2).copy()


BB = 8
NR = BB * N


def _feat_body(s_ref, wf_ref, bf_ref, hraw_ref, stats_ref):
    b = pl.program_id(0)
    x = s_ref[...].reshape(BB * T * N, D)
    hr = jnp.dot(x.astype(jnp.bfloat16), wf_ref[...].astype(jnp.bfloat16),
                 preferred_element_type=jnp.float32) + bf_ref[...]
    hraw_ref[...] = hr.reshape(BB, T, N, H)

    @pl.when(b == 0)
    def _():
        stats_ref[...] = jnp.zeros_like(stats_ref)

    stats_ref[0:1] += jnp.sum(hr, axis=0, keepdims=True)
    stats_ref[1:2] += jnp.sum(hr * hr, axis=0, keepdims=True)


def _stage_a(s, W_feat, bf):
    return pl.pallas_call(
        _feat_body,
        grid=(B // BB,),
        in_specs=[
            pl.BlockSpec((BB, T, N, D), lambda b: (b, 0, 0, 0)),
            pl.BlockSpec((D, H), lambda b: (0, 0)),
            pl.BlockSpec((1, H), lambda b: (0, 0)),
        ],
        out_specs=[
            pl.BlockSpec((BB, T, N, H), lambda b: (b, 0, 0, 0)),
            pl.BlockSpec((8, H), lambda b: (0, 0)),
        ],
        out_shape=[
            jax.ShapeDtypeStruct((B, T, N, H), jnp.float32),
            jax.ShapeDtypeStruct((8, H), jnp.float32),
        ],
    )(s, W_feat, bf)


def _score_body(hraw_ref, kflat_ref, stats_ref, gin_ref, bin_ref, wml_ref,
                bml_ref, mu_ref, lv_ref, st2_ref):
    b = pl.program_id(0)
    cnt = float(B * T * N)
    mean = stats_ref[0:1] / cnt
    var = stats_ref[1:2] / cnt - mean * mean
    std = jnp.sqrt(var + 1e-5)

    hr = hraw_ref[...].reshape(BB * T * N, H)
    hbn = (hr - mean) / std * gin_ref[...] + bin_ref[...]
    hml = jnp.dot(hbn.astype(jnp.bfloat16), wml_ref[...].astype(jnp.bfloat16),
                  preferred_element_type=jnp.float32) + bml_ref[...]
    hml4 = hml.reshape(BB, T, N, 2)
    hmuT = jnp.transpose(hml4[:, :, :, 0], (0, 2, 1)).reshape(NR, T)
    hlvT = jnp.transpose(hml4[:, :, :, 1], (0, 2, 1)).reshape(NR, T)

    kflat = kflat_ref[...].reshape(BB * N * MN, 1)
    blocal = lax.broadcasted_iota(jnp.int32, (BB * N * MN, 1), 0) // (N * MN)
    kglob = kflat + blocal * N
    onehot = (kglob == lax.broadcasted_iota(jnp.int32, (BB * N * MN, NR), 1)
              ).astype(jnp.float32)
    mu_nei = jnp.dot(onehot, hmuT, preferred_element_type=jnp.float32).reshape(NR, MN, T)
    lv_nei = jnp.dot(onehot, hlvT, preferred_element_type=jnp.float32).reshape(NR, MN, T)
    mu_raw = jnp.concatenate([hmuT.reshape(NR, 1, T), mu_nei], axis=1)
    lv_raw = jnp.concatenate([hlvT.reshape(NR, 1, T), lv_nei], axis=1)
    mu_ref[...] = mu_raw.reshape(BB, N, J, T)
    lv_ref[...] = lv_raw.reshape(BB, N, J, T)

    @pl.when(b == 0)
    def _():
        st2_ref[...] = jnp.zeros_like(st2_ref)

    def _nodesum(x):
        s = jnp.sum(jnp.sum(x, axis=2), axis=1).reshape(BB, N)
        return jnp.sum(s.T, axis=1, keepdims=True)

    st2_ref[:, 0:1] += _nodesum(mu_raw)
    st2_ref[:, 1:2] += _nodesum(mu_raw * mu_raw)
    st2_ref[:, 2:3] += _nodesum(lv_raw)
    st2_ref[:, 3:4] += _nodesum(lv_raw * lv_raw)


def _stage_b(hraw, kflat, stats, gin, bin_, wml, bml):
    return pl.pallas_call(
        _score_body,
        grid=(B // BB,),
        in_specs=[
            pl.BlockSpec((BB, T, N, H), lambda b: (b, 0, 0, 0)),
            pl.BlockSpec((BB, N * MN, 1), lambda b: (b, 0, 0)),
            pl.BlockSpec((8, H), lambda b: (0, 0)),
            pl.BlockSpec((1, H), lambda b: (0, 0)),
            pl.BlockSpec((1, H), lambda b: (0, 0)),
            pl.BlockSpec((H, 2), lambda b: (0, 0)),
            pl.BlockSpec((1, 2), lambda b: (0, 0)),
        ],
        out_specs=[
            pl.BlockSpec((BB, N, J, T), lambda b: (b, 0, 0, 0)),
            pl.BlockSpec((BB, N, J, T), lambda b: (b, 0, 0, 0)),
            pl.BlockSpec((N, 8), lambda b: (0, 0)),
        ],
        out_shape=[
            jax.ShapeDtypeStruct((B, N, J, T), jnp.float32),
            jax.ShapeDtypeStruct((B, N, J, T), jnp.float32),
            jax.ShapeDtypeStruct((N, 8), jnp.float32),
        ],
    )(hraw, kflat, stats, gin, bin_, wml, bml)


def _select_body(mu_ref, lv_ref, st2_ref, gmu_ref, bmu_ref, glv_ref, blv_ref,
                 eps_ref, nnei_ref, knei_ref, gidx_ref):
    b = pl.program_id(0)
    cntn = float(B * T * J)
    msum = st2_ref[:, 0:1]; msq = st2_ref[:, 1:2]
    lsum = st2_ref[:, 2:3]; lsq = st2_ref[:, 3:4]
    mmean = msum / cntn
    mstd = jnp.sqrt(msq / cntn - mmean * mmean + 1e-5)
    lmean = lsum / cntn
    lstd = jnp.sqrt(lsq / cntn - lmean * lmean + 1e-5)

    mm = mmean.reshape(1, N, 1, 1); ms = mstd.reshape(1, N, 1, 1)
    lm = lmean.reshape(1, N, 1, 1); ls = lstd.reshape(1, N, 1, 1)
    gm = gmu_ref[...].reshape(1, N, 1, 1); bm = bmu_ref[...].reshape(1, N, 1, 1)
    gl = glv_ref[...].reshape(1, N, 1, 1); bl = blv_ref[...].reshape(1, N, 1, 1)
    mu = jnp.tanh((mu_ref[...] - mm) / ms * gm + bm)
    logvar = jnp.clip((lv_ref[...] - lm) / ls * gl + bl, SIGMA_MIN, SIGMA_MAX)
    dis = (eps_ref[...] * jnp.exp(logvar) + mu).reshape(NR, J, T)

    t_sc = dis[:, 0, :]
    s_sc = dis[:, 1:, :]
    mrange = lax.broadcasted_iota(jnp.int32, (NR, MN, T), 1)
    nn = nnei_ref[...].reshape(NR, 1, 1)
    s_sc = jnp.where(mrange < nn, s_sc, -1e9)

    iota_t = lax.broadcasted_iota(jnp.int32, (NR, T), 1)
    iota_t3 = lax.broadcasted_iota(jnp.int32, (NR, MN, T), 2)
    iota_m = lax.broadcasted_iota(jnp.int32, (NR, MN), 1)
    row_ids = lax.broadcasted_iota(jnp.int32, (NR, 1), 0)
    n_ids = row_ids % N
    b_ids = b * BB + row_ids // N
    kneif = knei_ref[...].reshape(NR, MN).astype(jnp.float32)
    neg_inf = jnp.float32(-jnp.inf)

    ts = t_sc
    tsels = []
    for _ in range(TK):
        mx = jnp.max(ts, axis=1, keepdims=True)
        idx = jnp.min(jnp.where(ts >= mx, iota_t, T), axis=1, keepdims=True)
        tsels.append(idx)
        ts = jnp.where(iota_t == idx, neg_inf, ts)

    cols = []
    for ti in range(TK):
        tsel = tsels[ti]
        rowbase = (b_ids * T + tsel) * N
        cols.append(rowbase + n_ids)
        sel_t = (iota_t3 == tsel.reshape(NR, 1, 1)).astype(jnp.float32)
        ss = jnp.sum(s_sc * sel_t, axis=2)
        for _ in range(SK):
            mx = jnp.max(ss, axis=1, keepdims=True)
            midx = jnp.min(jnp.where(ss >= mx, iota_m, MN), axis=1, keepdims=True)
            hit = iota_m == midx
            ss = jnp.where(hit, neg_inf, ss)
            ksel = jnp.sum(jnp.where(hit, kneif, 0.0), axis=1, keepdims=True)
            cols.append(rowbase + ksel.astype(jnp.int32))
    gidx_ref[...] = jnp.concatenate(cols, axis=1).reshape(BB, N, SUB)


def _stage_c(mu_raw, lv_raw, st2, gmu, bmu, glv, blv, eps_t, nnei3, k_nei):
    return pl.pallas_call(
        _select_body,
        grid=(B // BB,),
        in_specs=[
            pl.BlockSpec((BB, N, J, T), lambda b: (b, 0, 0, 0)),
            pl.BlockSpec((BB, N, J, T), lambda b: (b, 0, 0, 0)),
            pl.BlockSpec((N, 8), lambda b: (0, 0)),
            pl.BlockSpec((N, 1), lambda b: (0, 0)),
            pl.BlockSpec((N, 1), lambda b: (0, 0)),
            pl.BlockSpec((N, 1), lambda b: (0, 0)),
            pl.BlockSpec((N, 1), lambda b: (0, 0)),
            pl.BlockSpec((BB, N, J, T), lambda b: (b, 0, 0, 0)),
            pl.BlockSpec((BB, N, 1), lambda b: (b, 0, 0)),
            pl.BlockSpec((BB, N, MN), lambda b: (b, 0, 0)),
        ],
        out_specs=pl.BlockSpec((BB, N, SUB), lambda b: (b, 0, 0)),
        out_shape=jax.ShapeDtypeStruct((B, N, SUB), jnp.int32),
    )(mu_raw, lv_raw, st2, gmu, bmu, glv, blv, eps_t, nnei3, k_nei)


_GW = 128


def _sc_gather(table, idx):
    idx2 = idx.reshape(1, ROWS)
    mesh = plsc.VectorSubcoreMesh(core_axis_name="core", subcore_axis_name="subcore")

    @functools.partial(
        pl.kernel,
        out_type=jax.ShapeDtypeStruct((ROWS, H), jnp.float32),
        mesh=mesh,
        compiler_params=pltpu.CompilerParams(use_tc_tiling_on_sc=False),
    )
    def k(x_hbm, i_hbm, o_hbm):
        def body(i_vmem, o_vmem):
            pltpu.sync_copy(x_hbm.at[i_vmem.at[0]], o_vmem)

        pltpu.emit_pipeline(
            body,
            grid=(ROWS // _GW,),
            in_specs=[pl.BlockSpec((1, _GW), index_map=lambda i: (0, i))],
            out_specs=[pl.BlockSpec((_GW, H), index_map=lambda i: (i, 0))],
            core_axis_name=("core", "subcore"),
            dimension_semantics=(pltpu.PARALLEL,),
        )(i_hbm, o_hbm)

    return k(table, idx2)


def _gnn_body(sub_ref, stats_ref, gin_ref, bin_ref, w1_ref, b1_ref, w2_ref,
              b2_ref, w3_ref, b3_ref, out_ref):
    cnt = float(B * T * N)
    mean = stats_ref[0:1] / cnt
    var = stats_ref[1:2] / cnt - mean * mean
    std = jnp.sqrt(var + 1e-5)

    x = (sub_ref[...].reshape(BB * N * SUB, H)
         - mean) / std * gin_ref[...] + bin_ref[...]
    pools = []
    for w_ref, b_ref in ((w1_ref, b1_ref), (w2_ref, b2_ref), (w3_ref, b3_ref)):
        x3 = x.reshape(NR, SUB, H)
        agg = jnp.mean(x3, axis=1, keepdims=True)
        xa = (x3 + agg).reshape(NR * SUB, H)
        x = jnp.maximum(
            jnp.dot(xa.astype(jnp.bfloat16), w_ref[...].astype(jnp.bfloat16),
                    preferred_element_type=jnp.float32) + b_ref[...],
            0.0)
        pools.append(jnp.mean(x.reshape(NR, SUB, H), axis=1))
    out_ref[...] = jnp.concatenate(pools, axis=1).reshape(BB, N, 3 * H)


def _stage_e(sub, stats, gin, bin_, Wg1, bg1, Wg2, bg2, Wg3, bg3):
    wspec = pl.BlockSpec((H, H), lambda b: (0, 0))
    bspec = pl.BlockSpec((1, H), lambda b: (0, 0))
    return pl.pallas_call(
        _gnn_body,
        grid=(B // BB,),
        in_specs=[
            pl.BlockSpec((BB, N * SUB, H), lambda b: (b, 0, 0)),
            pl.BlockSpec((8, H), lambda b: (0, 0)),
            bspec, bspec, wspec, bspec, wspec, bspec, wspec, bspec,
        ],
        out_specs=pl.BlockSpec((BB, N, 3 * H), lambda b: (b, 0, 0)),
        out_shape=jax.ShapeDtypeStruct((B, N, 3 * H), jnp.float32),
    )(sub, stats, gin, bin_, Wg1, bg1, Wg2, bg2, Wg3, bg3)


def kernel(s, k_nei, n_nei, W_feat, b_feat, g_in, beta_in, W_mu, b_mu, g_mu,
           beta_mu, W_lv, b_lv, g_lv, beta_lv, Wg1, bg1, Wg2, bg2, Wg3, bg3):
    bf = b_feat.reshape(1, H)
    gin = g_in.reshape(1, H)
    bin_ = beta_in.reshape(1, H)
    wml = jnp.concatenate([W_mu, W_lv], axis=1)
    bml = jnp.concatenate([b_mu, b_lv]).reshape(1, 2)
    gmu = g_mu.reshape(N, 1)
    bmu = beta_mu.reshape(N, 1)
    glv = g_lv.reshape(N, 1)
    blv = beta_lv.reshape(N, 1)
    eps_t = jnp.asarray(_EPS_T)
    nnei3 = n_nei.reshape(B, N, 1)
    kflat = k_nei.reshape(B, N * MN, 1)

    hraw, stats = _stage_a(s, W_feat, bf)
    mu_raw, lv_raw, st2 = _stage_b(hraw, kflat, stats, gin, bin_, wml, bml)
    gidx = _stage_c(mu_raw, lv_raw, st2, gmu, bmu, glv, blv, eps_t, nnei3, k_nei)
    sub = _sc_gather(hraw.reshape(TBL, H), gidx.reshape(ROWS))
    return _stage_e(sub.reshape(B, N * SUB, H), stats, gin, bin_,
                    Wg1, bg1.reshape(1, H), Wg2, bg2.reshape(1, H),
                    Wg3, bg3.reshape(1, H))
